# baseline (speedup 1.0000x reference)
import functools

import jax
import jax.numpy as jnp
from jax import lax
from jax.experimental import pallas as pl
from jax.experimental.pallas import tpu as pltpu
from jax.experimental.pallas import tpu_sc as plsc

N = 50000
E = 800000
R = 8
D = 64
H = 64
O = 32

NC = 2
NS = 16
LANES = 16

NP = 50048
DUMP = N
EP = 819200
CH = 400
CHA = 320
TILE_ROWS = NP // NS

_MESH = plsc.VectorSubcoreMesh(
    core_axis_name="c", subcore_axis_name="s", num_cores=NC, num_subcores=NS
)
_SC_PARAMS = pltpu.CompilerParams(
    use_tc_tiling_on_sc=False, needs_layout_passes=False
)


def _lane_splat(v, j):
    return v.at[jnp.full((LANES,), j, jnp.int32)].get(mode="promise_in_bounds")


def _count_body(dst_h, et_h, z_h, cnt0_h, cnt1_h, dstv, etv, oh, acc):
    c = lax.axis_index("c")
    s = lax.axis_index("s")
    rbase = pl.multiple_of(s * TILE_ROWS, 8)
    rows_sl = pl.ds(rbase, TILE_ROWS)
    pltpu.sync_copy(z_h.at[rows_sl], acc.at[rows_sl])
    plsc.subcore_barrier()

    per_tile = EP // (NC * NS)
    ebase = (c * NS + s) * per_tile
    iota = lax.iota(jnp.int32, LANES)

    @pl.loop(0, per_tile // CH)
    def _chunk(ch):
        base = ebase + ch * CH
        pltpu.sync_copy(dst_h.at[pl.ds(base, CH)], dstv)
        pltpu.sync_copy(et_h.at[pl.ds(base, CH)], etv)

        @pl.loop(0, CH // LANES)
        def _group(g):
            o = pl.multiple_of(g * LANES, LANES)
            tv = etv[pl.ds(o, LANES)]
            for j in range(LANES):
                oh[o + j, :] = jnp.where(iota == _lane_splat(tv, j), 1.0, 0.0)

        pltpu.sync_copy(oh, acc.at[dstv], add=True)

    plsc.subcore_barrier()

    @pl.when(c == 0)
    def _():
        pltpu.sync_copy(acc.at[rows_sl], cnt0_h.at[rows_sl])

    @pl.when(c == 1)
    def _():
        pltpu.sync_copy(acc.at[rows_sl], cnt1_h.at[rows_sl])


_count_kernel = pl.kernel(
    _count_body,
    out_type=[
        jax.ShapeDtypeStruct((NP, 16), jnp.float32),
        jax.ShapeDtypeStruct((NP, 16), jnp.float32),
    ],
    mesh=_MESH,
    scratch_types=[
        pltpu.VMEM((CH,), jnp.int32),
        pltpu.VMEM((CH,), jnp.int32),
        pltpu.VMEM((CH, 16), jnp.float32),
        pltpu.VMEM_SHARED((NP, 16), jnp.float32),
    ],
    compiler_params=_SC_PARAMS,
)


def _sedge_body(dst_h, et_h, s_h, se_h,
                dstv0, dstv1, etv0, etv1, srows0, srows1, sebuf0, sebuf1,
                sem0, sem1):
    dstv = (dstv0, dstv1)
    etv = (etv0, etv1)
    srows = (srows0, srows1)
    sebuf = (sebuf0, sebuf1)
    sem = (sem0, sem1)
    c = lax.axis_index("c")
    s = lax.axis_index("s")
    per_tile = EP // (NC * NS)
    ebase = (c * NS + s) * per_tile
    iota = lax.iota(jnp.int32, LANES)

    def load(i, b):
        base = ebase + i * CH
        pltpu.sync_copy(dst_h.at[pl.ds(base, CH)], dstv[b])
        pltpu.sync_copy(et_h.at[pl.ds(base, CH)], etv[b])
        pltpu.async_copy(s_h.at[dstv[b]], srows[b], sem[b])

    def extract_store(i, b):
        pltpu.make_async_copy(s_h.at[dstv[b]], srows[b], sem[b]).wait()

        @pl.loop(0, CH // LANES)
        def _group(g):
            o = pl.multiple_of(g * LANES, LANES)
            tv = etv[b][pl.ds(o, LANES)]
            sebuf[b][pl.ds(o, LANES)] = plsc.load_gather(srows[b], [o + iota, tv])

        base = ebase + i * CH
        pltpu.sync_copy(sebuf[b], se_h.at[pl.ds(base, CH)])

    @pl.loop(0, per_tile // (2 * CH))
    def _pair(it):
        i0 = it * 2
        load(i0, 0)
        load(i0 + 1, 1)
        extract_store(i0, 0)
        extract_store(i0 + 1, 1)


_sedge_kernel = pl.kernel(
    _sedge_body,
    out_type=jax.ShapeDtypeStruct((EP,), jnp.float32),
    mesh=_MESH,
    scratch_types=[
        pltpu.VMEM((CH,), jnp.int32),
        pltpu.VMEM((CH,), jnp.int32),
        pltpu.VMEM((CH,), jnp.int32),
        pltpu.VMEM((CH,), jnp.int32),
        pltpu.VMEM((CH, 16), jnp.float32),
        pltpu.VMEM((CH, 16), jnp.float32),
        pltpu.VMEM((CH,), jnp.float32),
        pltpu.VMEM((CH,), jnp.float32),
        pltpu.SemaphoreType.DMA,
        pltpu.SemaphoreType.DMA,
    ],
    compiler_params=_SC_PARAMS,
)


def _agg_body(W, dim_split, src_h, dst_h, et_h, x0_h, x1_h, se_h, z_h,
              y0_h, y1_h,
              srcv0, srcv1, dstv0, dstv1, etv0, etv1, gidx0, gidx1,
              sebuf0, sebuf1, rows0, rows1, acc, semx0, semx1):
    srcv = (srcv0, srcv1)
    dstv = (dstv0, dstv1)
    etv = (etv0, etv1)
    gidx = (gidx0, gidx1)
    sebuf = (sebuf0, sebuf1)
    rows = (rows0, rows1)
    semx = (semx0, semx1)
    c = lax.axis_index("c")
    s = lax.axis_index("s")
    rbase = pl.multiple_of(s * TILE_ROWS, 8)
    rows_sl = pl.ds(rbase, TILE_ROWS)
    pltpu.sync_copy(z_h.at[rows_sl], acc.at[rows_sl])
    plsc.subcore_barrier()

    if dim_split:
        per_tile = EP // NS
        ebase = s * per_tile
    else:
        per_tile = EP // (NC * NS)
        ebase = (c * NS + s) * per_tile
    iota = lax.iota(jnp.int32, LANES)

    def load_linear(i, b):
        base = ebase + i * CHA
        pltpu.sync_copy(src_h.at[pl.ds(base, CHA)], srcv[b])
        pltpu.sync_copy(dst_h.at[pl.ds(base, CHA)], dstv[b])
        pltpu.sync_copy(et_h.at[pl.ds(base, CHA)], etv[b])
        pltpu.sync_copy(se_h.at[pl.ds(base, CHA)], sebuf[b])

        @pl.loop(0, CHA // LANES)
        def _mkidx(g):
            o = pl.multiple_of(g * LANES, LANES)
            gidx[b][pl.ds(o, LANES)] = (
                etv[b][pl.ds(o, LANES)] * N + srcv[b][pl.ds(o, LANES)]
            )

    def wait_gather(b):
        pltpu.make_async_copy(x0_h.at[gidx[b]], rows[b], semx[b]).wait()

    def scale_scatter(b):
        @pl.loop(0, CHA // LANES)
        def _scale(g):
            o = pl.multiple_of(g * LANES, LANES)
            sval = sebuf[b][pl.ds(o, LANES)]
            for j in range(LANES):
                sj = _lane_splat(sval, j)
                for q in range(W // LANES):
                    qs = pl.ds(q * LANES, LANES)
                    rows[b][o + j, qs] = rows[b][o + j, qs] * sj

        pltpu.sync_copy(rows[b], acc.at[dstv[b]], add=True)

    def start(b):
        if dim_split:
            @pl.when(c == 0)
            def _():
                pltpu.async_copy(x0_h.at[gidx[b]], rows[b], semx[b])

            @pl.when(c == 1)
            def _():
                pltpu.async_copy(x1_h.at[gidx[b]], rows[b], semx[b])
        else:
            pltpu.async_copy(x0_h.at[gidx[b]], rows[b], semx[b])

    @pl.loop(0, per_tile // (2 * CHA))
    def _pair(it):
        i0 = it * 2
        load_linear(i0, 0)
        start(0)
        load_linear(i0 + 1, 1)
        wait_gather(0)
        start(1)
        scale_scatter(0)
        wait_gather(1)
        scale_scatter(1)

    plsc.subcore_barrier()

    @pl.when(c == 0)
    def _():
        pltpu.sync_copy(acc.at[rows_sl], y0_h.at[rows_sl])

    @pl.when(c == 1)
    def _():
        pltpu.sync_copy(acc.at[rows_sl], y1_h.at[rows_sl])


def _make_agg_kernel(W, dim_split):
    return pl.kernel(
        functools.partial(_agg_body, W, dim_split),
        out_type=[
            jax.ShapeDtypeStruct((NP, W), jnp.float32),
            jax.ShapeDtypeStruct((NP, W), jnp.float32),
        ],
        mesh=_MESH,
        scratch_types=(
            [pltpu.VMEM((CHA,), jnp.int32)] * 8
            + [pltpu.VMEM((CHA,), jnp.float32)] * 2
            + [pltpu.VMEM((CHA, W), jnp.float32)] * 2
            + [
                pltpu.VMEM_SHARED((NP, W), jnp.float32),
                pltpu.SemaphoreType.DMA,
                pltpu.SemaphoreType.DMA,
            ]
        ),
        compiler_params=_SC_PARAMS,
    )


_agg_l1 = _make_agg_kernel(32, True)
_agg_l2 = _make_agg_kernel(32, False)


_BN = 2000


def _xt2_body(x_ref, w_ref, x0_ref, x1_ref):
    p = jnp.dot(x_ref[...], w_ref[0], preferred_element_type=jnp.float32)
    half = p.shape[1] // 2
    x0_ref[...] = p[:, :half]
    x1_ref[...] = p[:, half:]


def _xt1_body(x_ref, w_ref, x0_ref):
    x0_ref[...] = jnp.dot(x_ref[...], w_ref[0],
                          preferred_element_type=jnp.float32)


def _xtables(x, w, dout, split):
    nb = N // _BN
    wout = dout // 2 if split else dout
    nout = 2 if split else 1
    out_spec = pl.BlockSpec((_BN, wout), lambda t, i: (t * (N // _BN) + i, 0))
    res = pl.pallas_call(
        _xt2_body if split else _xt1_body,
        grid=(R, nb),
        in_specs=[
            pl.BlockSpec((_BN, x.shape[1]), lambda t, i: (i, 0)),
            pl.BlockSpec((1, w.shape[1], dout), lambda t, i: (t, 0, 0)),
        ],
        out_specs=[out_spec] * nout,
        out_shape=[jax.ShapeDtypeStruct((R * N, wout), jnp.float32)] * nout,
    )(x, w)
    return res


def _recip_body(c0_ref, c1_ref, s_ref):
    s_ref[...] = 1.0 / jnp.clip(c0_ref[...] + c1_ref[...], 1.0, None)


def _recip(cnt0, cnt1):
    r = pl.pallas_call(
        _recip_body,
        out_shape=jax.ShapeDtypeStruct((NP * 16 // 128, 128), jnp.float32),
    )(cnt0.reshape(NP * 16 // 128, 128), cnt1.reshape(NP * 16 // 128, 128))
    return r.reshape(NP, 16)


def _dense_body(act, concat, x_ref, r_ref, b_ref, y0_ref, y1_ref, o_ref):
    t = jnp.dot(x_ref[...], r_ref[...], preferred_element_type=jnp.float32)
    if concat:
        t = t + b_ref[...] + jnp.concatenate([y0_ref[...], y1_ref[...]], axis=1)
    else:
        t = t + b_ref[...] + y0_ref[...] + y1_ref[...]
    o_ref[...] = act(t)


def _dense(x, root, b, y0, y1, act, concat):
    dout = root.shape[1]
    w = dout // 2 if concat else dout
    nb = N // _BN
    return pl.pallas_call(
        functools.partial(_dense_body, act, concat),
        grid=(nb,),
        in_specs=[
            pl.BlockSpec((_BN, x.shape[1]), lambda i: (i, 0)),
            pl.BlockSpec(root.shape, lambda i: (0, 0)),
            pl.BlockSpec((1, dout), lambda i: (0, 0)),
            pl.BlockSpec((_BN, w), lambda i: (i, 0)),
            pl.BlockSpec((_BN, w), lambda i: (i, 0)),
        ],
        out_specs=pl.BlockSpec((_BN, dout), lambda i: (i, 0)),
        out_shape=jax.ShapeDtypeStruct((N, dout), jnp.float32),
    )(x, root, b.reshape(1, dout), y0, y1)


def kernel(emb, w1, root1, b1, w2, root2, b2, edge_index, edge_type):
    pad = EP - E
    src = jnp.concatenate([edge_index[0], jnp.zeros((pad,), jnp.int32)])
    dst = jnp.concatenate([edge_index[1], jnp.full((pad,), DUMP, jnp.int32)])
    et = jnp.concatenate([edge_type, jnp.zeros((pad,), jnp.int32)])

    z16 = jnp.zeros((NP, 16), jnp.float32)
    z32 = jnp.zeros((NP, 32), jnp.float32)

    cnt0, cnt1 = _count_kernel(dst, et, z16)
    s_tab = _recip(cnt0, cnt1)
    s_e = _sedge_kernel(dst, et, s_tab)

    x10, x11 = _xtables(emb, w1, H, True)
    y10, y11 = _agg_l1(src, dst, et, x10, x11, s_e, z32)
    h = _dense(emb, root1, b1, y10[:N], y11[:N],
               lambda v: jnp.maximum(v, 0.0), True)

    (x2,) = _xtables(h, w2, O, False)
    y20, y21 = _agg_l2(src, dst, et, x2, x2, s_e, z32)
    out = _dense(h, root2, b2, y20[:N], y21[:N], jax.nn.sigmoid, False)
    return out


# speedup vs baseline: 1.0781x; 1.0781x over previous
import functools

import jax
import jax.numpy as jnp
from jax import lax
from jax.experimental import pallas as pl
from jax.experimental.pallas import tpu as pltpu
from jax.experimental.pallas import tpu_sc as plsc

N = 50000
E = 800000
R = 8
D = 64
H = 64
O = 32

NC = 2
NS = 16
LANES = 16

NP = 50048
DUMP = N
EP = 819200
CH = 400
CHA = 320
TILE_ROWS = NP // NS

_MESH = plsc.VectorSubcoreMesh(
    core_axis_name="c", subcore_axis_name="s", num_cores=NC, num_subcores=NS
)
_SC_PARAMS = pltpu.CompilerParams(
    use_tc_tiling_on_sc=False, needs_layout_passes=False
)


def _lane_splat(v, j):
    return v.at[jnp.full((LANES,), j, jnp.int32)].get(mode="promise_in_bounds")


def _count_body(dst_h, et_h, z_h, cnt0_h, cnt1_h, dstv, etv, oh, acc):
    c = lax.axis_index("c")
    s = lax.axis_index("s")
    rbase = pl.multiple_of(s * TILE_ROWS, 8)
    rows_sl = pl.ds(rbase, TILE_ROWS)
    pltpu.sync_copy(z_h.at[rows_sl], acc.at[rows_sl])
    plsc.subcore_barrier()

    per_tile = EP // (NC * NS)
    ebase = (c * NS + s) * per_tile
    iota = lax.iota(jnp.int32, LANES)

    @pl.loop(0, per_tile // CH)
    def _chunk(ch):
        base = ebase + ch * CH
        pltpu.sync_copy(dst_h.at[pl.ds(base, CH)], dstv)
        pltpu.sync_copy(et_h.at[pl.ds(base, CH)], etv)

        @pl.loop(0, CH // LANES)
        def _group(g):
            o = pl.multiple_of(g * LANES, LANES)
            tv = etv[pl.ds(o, LANES)]
            for j in range(LANES):
                oh[o + j, :] = jnp.where(iota == _lane_splat(tv, j), 1.0, 0.0)

        pltpu.sync_copy(oh, acc.at[dstv], add=True)

    plsc.subcore_barrier()

    @pl.when(c == 0)
    def _():
        pltpu.sync_copy(acc.at[rows_sl], cnt0_h.at[rows_sl])

    @pl.when(c == 1)
    def _():
        pltpu.sync_copy(acc.at[rows_sl], cnt1_h.at[rows_sl])


_count_kernel = pl.kernel(
    _count_body,
    out_type=[
        jax.ShapeDtypeStruct((NP, 16), jnp.float32),
        jax.ShapeDtypeStruct((NP, 16), jnp.float32),
    ],
    mesh=_MESH,
    scratch_types=[
        pltpu.VMEM((CH,), jnp.int32),
        pltpu.VMEM((CH,), jnp.int32),
        pltpu.VMEM((CH, 16), jnp.float32),
        pltpu.VMEM_SHARED((NP, 16), jnp.float32),
    ],
    compiler_params=_SC_PARAMS,
)


def _sedge_body(dst_h, et_h, s_h, se_h,
                dstv0, dstv1, etv0, etv1, srows0, srows1, sebuf0, sebuf1,
                sem0, sem1):
    dstv = (dstv0, dstv1)
    etv = (etv0, etv1)
    srows = (srows0, srows1)
    sebuf = (sebuf0, sebuf1)
    sem = (sem0, sem1)
    c = lax.axis_index("c")
    s = lax.axis_index("s")
    per_tile = EP // (NC * NS)
    ebase = (c * NS + s) * per_tile
    iota = lax.iota(jnp.int32, LANES)

    def load(i, b):
        base = ebase + i * CH
        pltpu.sync_copy(dst_h.at[pl.ds(base, CH)], dstv[b])
        pltpu.sync_copy(et_h.at[pl.ds(base, CH)], etv[b])
        pltpu.async_copy(s_h.at[dstv[b]], srows[b], sem[b])

    def extract_store(i, b):
        pltpu.make_async_copy(s_h.at[dstv[b]], srows[b], sem[b]).wait()

        @pl.loop(0, CH // LANES)
        def _group(g):
            o = pl.multiple_of(g * LANES, LANES)
            tv = etv[b][pl.ds(o, LANES)]
            sebuf[b][pl.ds(o, LANES)] = plsc.load_gather(srows[b], [o + iota, tv])

        base = ebase + i * CH
        pltpu.sync_copy(sebuf[b], se_h.at[pl.ds(base, CH)])

    @pl.loop(0, per_tile // (2 * CH))
    def _pair(it):
        i0 = it * 2
        load(i0, 0)
        load(i0 + 1, 1)
        extract_store(i0, 0)
        extract_store(i0 + 1, 1)


_sedge_kernel = pl.kernel(
    _sedge_body,
    out_type=jax.ShapeDtypeStruct((EP,), jnp.float32),
    mesh=_MESH,
    scratch_types=[
        pltpu.VMEM((CH,), jnp.int32),
        pltpu.VMEM((CH,), jnp.int32),
        pltpu.VMEM((CH,), jnp.int32),
        pltpu.VMEM((CH,), jnp.int32),
        pltpu.VMEM((CH, 16), jnp.float32),
        pltpu.VMEM((CH, 16), jnp.float32),
        pltpu.VMEM((CH,), jnp.float32),
        pltpu.VMEM((CH,), jnp.float32),
        pltpu.SemaphoreType.DMA,
        pltpu.SemaphoreType.DMA,
    ],
    compiler_params=_SC_PARAMS,
)


def _agg_body(W, dim_split, src_h, dst_h, et_h, x0_h, x1_h, se_h, z_h,
              y0_h, y1_h,
              srcv0, srcv1, dstv0, dstv1, etv0, etv1, gidx0, gidx1,
              sebuf0, sebuf1, rows0, rows1, acc, semx0, semx1):
    srcv = (srcv0, srcv1)
    dstv = (dstv0, dstv1)
    etv = (etv0, etv1)
    gidx = (gidx0, gidx1)
    sebuf = (sebuf0, sebuf1)
    rows = (rows0, rows1)
    semx = (semx0, semx1)
    c = lax.axis_index("c")
    s = lax.axis_index("s")
    rbase = pl.multiple_of(s * TILE_ROWS, 8)
    rows_sl = pl.ds(rbase, TILE_ROWS)
    pltpu.sync_copy(z_h.at[rows_sl], acc.at[rows_sl])
    plsc.subcore_barrier()

    if dim_split:
        per_tile = EP // NS
        ebase = s * per_tile
    else:
        per_tile = EP // (NC * NS)
        ebase = (c * NS + s) * per_tile
    iota = lax.iota(jnp.int32, LANES)

    def load_linear(i, b):
        base = ebase + i * CHA
        pltpu.sync_copy(src_h.at[pl.ds(base, CHA)], srcv[b])
        pltpu.sync_copy(dst_h.at[pl.ds(base, CHA)], dstv[b])
        pltpu.sync_copy(et_h.at[pl.ds(base, CHA)], etv[b])
        pltpu.sync_copy(se_h.at[pl.ds(base, CHA)], sebuf[b])

        @pl.loop(0, CHA // LANES)
        def _mkidx(g):
            o = pl.multiple_of(g * LANES, LANES)
            gidx[b][pl.ds(o, LANES)] = (
                etv[b][pl.ds(o, LANES)] * N + srcv[b][pl.ds(o, LANES)]
            )

    def wait_gather(b):
        pltpu.make_async_copy(x0_h.at[gidx[b]], rows[b], semx[b]).wait()

    def scale_scatter(b):
        @pl.loop(0, CHA // LANES)
        def _scale(g):
            o = pl.multiple_of(g * LANES, LANES)
            sval = sebuf[b][pl.ds(o, LANES)]
            for j in range(LANES):
                sj = _lane_splat(sval, j)
                for q in range(W // LANES):
                    qs = pl.ds(q * LANES, LANES)
                    rows[b][o + j, qs] = rows[b][o + j, qs] * sj

        pltpu.sync_copy(rows[b], acc.at[dstv[b]], add=True)

    def start(b):
        if dim_split:
            @pl.when(c == 0)
            def _():
                pltpu.async_copy(x0_h.at[gidx[b]], rows[b], semx[b])

            @pl.when(c == 1)
            def _():
                pltpu.async_copy(x1_h.at[gidx[b]], rows[b], semx[b])
        else:
            pltpu.async_copy(x0_h.at[gidx[b]], rows[b], semx[b])

    npairs = per_tile // (2 * CHA)
    load_linear(0, 0)
    start(0)

    @pl.loop(0, npairs)
    def _pair(it):
        i0 = it * 2
        load_linear(i0 + 1, 1)
        wait_gather(0)
        start(1)
        scale_scatter(0)

        @pl.when(it + 1 < npairs)
        def _prefetch():
            load_linear(i0 + 2, 0)
            start(0)

        wait_gather(1)
        scale_scatter(1)

    plsc.subcore_barrier()

    @pl.when(c == 0)
    def _():
        pltpu.sync_copy(acc.at[rows_sl], y0_h.at[rows_sl])

    @pl.when(c == 1)
    def _():
        pltpu.sync_copy(acc.at[rows_sl], y1_h.at[rows_sl])


def _make_agg_kernel(W, dim_split):
    return pl.kernel(
        functools.partial(_agg_body, W, dim_split),
        out_type=[
            jax.ShapeDtypeStruct((NP, W), jnp.float32),
            jax.ShapeDtypeStruct((NP, W), jnp.float32),
        ],
        mesh=_MESH,
        scratch_types=(
            [pltpu.VMEM((CHA,), jnp.int32)] * 8
            + [pltpu.VMEM((CHA,), jnp.float32)] * 2
            + [pltpu.VMEM((CHA, W), jnp.float32)] * 2
            + [
                pltpu.VMEM_SHARED((NP, W), jnp.float32),
                pltpu.SemaphoreType.DMA,
                pltpu.SemaphoreType.DMA,
            ]
        ),
        compiler_params=_SC_PARAMS,
    )


_agg_l1 = _make_agg_kernel(32, True)
_agg_l2 = _make_agg_kernel(32, False)


_BN = 2000


def _xt2_body(x_ref, w_ref, x0_ref, x1_ref):
    p = jnp.dot(x_ref[...], w_ref[0], preferred_element_type=jnp.float32)
    half = p.shape[1] // 2
    x0_ref[...] = p[:, :half]
    x1_ref[...] = p[:, half:]


def _xt1_body(x_ref, w_ref, x0_ref):
    x0_ref[...] = jnp.dot(x_ref[...], w_ref[0],
                          preferred_element_type=jnp.float32)


def _xtables(x, w, dout, split):
    nb = N // _BN
    wout = dout // 2 if split else dout
    nout = 2 if split else 1
    out_spec = pl.BlockSpec((_BN, wout), lambda t, i: (t * (N // _BN) + i, 0))
    res = pl.pallas_call(
        _xt2_body if split else _xt1_body,
        grid=(R, nb),
        in_specs=[
            pl.BlockSpec((_BN, x.shape[1]), lambda t, i: (i, 0)),
            pl.BlockSpec((1, w.shape[1], dout), lambda t, i: (t, 0, 0)),
        ],
        out_specs=[out_spec] * nout,
        out_shape=[jax.ShapeDtypeStruct((R * N, wout), jnp.float32)] * nout,
    )(x, w)
    return res


def _recip_body(c0_ref, c1_ref, s_ref):
    s_ref[...] = 1.0 / jnp.clip(c0_ref[...] + c1_ref[...], 1.0, None)


def _recip(cnt0, cnt1):
    r = pl.pallas_call(
        _recip_body,
        out_shape=jax.ShapeDtypeStruct((NP * 16 // 128, 128), jnp.float32),
    )(cnt0.reshape(NP * 16 // 128, 128), cnt1.reshape(NP * 16 // 128, 128))
    return r.reshape(NP, 16)


def _dense_body(act, concat, x_ref, r_ref, b_ref, y0_ref, y1_ref, o_ref):
    t = jnp.dot(x_ref[...], r_ref[...], preferred_element_type=jnp.float32)
    if concat:
        t = t + b_ref[...] + jnp.concatenate([y0_ref[...], y1_ref[...]], axis=1)
    else:
        t = t + b_ref[...] + y0_ref[...] + y1_ref[...]
    o_ref[...] = act(t)


def _dense(x, root, b, y0, y1, act, concat):
    dout = root.shape[1]
    w = dout // 2 if concat else dout
    nb = N // _BN
    return pl.pallas_call(
        functools.partial(_dense_body, act, concat),
        grid=(nb,),
        in_specs=[
            pl.BlockSpec((_BN, x.shape[1]), lambda i: (i, 0)),
            pl.BlockSpec(root.shape, lambda i: (0, 0)),
            pl.BlockSpec((1, dout), lambda i: (0, 0)),
            pl.BlockSpec((_BN, w), lambda i: (i, 0)),
            pl.BlockSpec((_BN, w), lambda i: (i, 0)),
        ],
        out_specs=pl.BlockSpec((_BN, dout), lambda i: (i, 0)),
        out_shape=jax.ShapeDtypeStruct((N, dout), jnp.float32),
    )(x, root, b.reshape(1, dout), y0, y1)


def kernel(emb, w1, root1, b1, w2, root2, b2, edge_index, edge_type):
    pad = EP - E
    src = jnp.concatenate([edge_index[0], jnp.zeros((pad,), jnp.int32)])
    dst = jnp.concatenate([edge_index[1], jnp.full((pad,), DUMP, jnp.int32)])
    et = jnp.concatenate([edge_type, jnp.zeros((pad,), jnp.int32)])

    z16 = jnp.zeros((NP, 16), jnp.float32)
    z32 = jnp.zeros((NP, 32), jnp.float32)

    cnt0, cnt1 = _count_kernel(dst, et, z16)
    s_tab = _recip(cnt0, cnt1)
    s_e = _sedge_kernel(dst, et, s_tab)

    x10, x11 = _xtables(emb, w1, H, True)
    y10, y11 = _agg_l1(src, dst, et, x10, x11, s_e, z32)
    h = _dense(emb, root1, b1, y10[:N], y11[:N],
               lambda v: jnp.maximum(v, 0.0), True)

    (x2,) = _xtables(h, w2, O, False)
    y20, y21 = _agg_l2(src, dst, et, x2, x2, s_e, z32)
    out = _dense(h, root2, b2, y20[:N], y21[:N], jax.nn.sigmoid, False)
    return out


# speedup vs baseline: 1.0994x; 1.0198x over previous
import functools

import jax
import jax.numpy as jnp
from jax import lax
from jax.experimental import pallas as pl
from jax.experimental.pallas import tpu as pltpu
from jax.experimental.pallas import tpu_sc as plsc

N = 50000
E = 800000
R = 8
D = 64
H = 64
O = 32

NC = 2
NS = 16
LANES = 16

NP = 50048
DUMP = N
EP = 819200
CH = 400
CHA = 400
TILE_ROWS = NP // NS

_MESH = plsc.VectorSubcoreMesh(
    core_axis_name="c", subcore_axis_name="s", num_cores=NC, num_subcores=NS
)
_SC_PARAMS = pltpu.CompilerParams(
    use_tc_tiling_on_sc=False, needs_layout_passes=False
)


def _lane_splat(v, j):
    return v.at[jnp.full((LANES,), j, jnp.int32)].get(mode="promise_in_bounds")


def _count_body(dst_h, et_h, z_h, cnt0_h, cnt1_h, dstv, etv, oh, acc):
    c = lax.axis_index("c")
    s = lax.axis_index("s")
    rbase = pl.multiple_of(s * TILE_ROWS, 8)
    rows_sl = pl.ds(rbase, TILE_ROWS)
    pltpu.sync_copy(z_h.at[rows_sl], acc.at[rows_sl])
    plsc.subcore_barrier()

    per_tile = EP // (NC * NS)
    ebase = (c * NS + s) * per_tile
    iota = lax.iota(jnp.int32, LANES)

    @pl.loop(0, per_tile // CH)
    def _chunk(ch):
        base = ebase + ch * CH
        pltpu.sync_copy(dst_h.at[pl.ds(base, CH)], dstv)
        pltpu.sync_copy(et_h.at[pl.ds(base, CH)], etv)

        @pl.loop(0, CH // LANES)
        def _group(g):
            o = pl.multiple_of(g * LANES, LANES)
            tv = etv[pl.ds(o, LANES)]
            for j in range(LANES):
                oh[o + j, :] = jnp.where(iota == _lane_splat(tv, j), 1.0, 0.0)

        pltpu.sync_copy(oh, acc.at[dstv], add=True)

    plsc.subcore_barrier()

    @pl.when(c == 0)
    def _():
        pltpu.sync_copy(acc.at[rows_sl], cnt0_h.at[rows_sl])

    @pl.when(c == 1)
    def _():
        pltpu.sync_copy(acc.at[rows_sl], cnt1_h.at[rows_sl])


_count_kernel = pl.kernel(
    _count_body,
    out_type=[
        jax.ShapeDtypeStruct((NP, 16), jnp.float32),
        jax.ShapeDtypeStruct((NP, 16), jnp.float32),
    ],
    mesh=_MESH,
    scratch_types=[
        pltpu.VMEM((CH,), jnp.int32),
        pltpu.VMEM((CH,), jnp.int32),
        pltpu.VMEM((CH, 16), jnp.float32),
        pltpu.VMEM_SHARED((NP, 16), jnp.float32),
    ],
    compiler_params=_SC_PARAMS,
)


def _sedge_body(dst_h, et_h, s_h, se_h,
                dstv0, dstv1, etv0, etv1, srows0, srows1, sebuf0, sebuf1,
                sem0, sem1):
    dstv = (dstv0, dstv1)
    etv = (etv0, etv1)
    srows = (srows0, srows1)
    sebuf = (sebuf0, sebuf1)
    sem = (sem0, sem1)
    c = lax.axis_index("c")
    s = lax.axis_index("s")
    per_tile = EP // (NC * NS)
    ebase = (c * NS + s) * per_tile
    iota = lax.iota(jnp.int32, LANES)

    def load(i, b):
        base = ebase + i * CH
        pltpu.sync_copy(dst_h.at[pl.ds(base, CH)], dstv[b])
        pltpu.sync_copy(et_h.at[pl.ds(base, CH)], etv[b])
        pltpu.async_copy(s_h.at[dstv[b]], srows[b], sem[b])

    def extract_store(i, b):
        pltpu.make_async_copy(s_h.at[dstv[b]], srows[b], sem[b]).wait()

        @pl.loop(0, CH // LANES)
        def _group(g):
            o = pl.multiple_of(g * LANES, LANES)
            tv = etv[b][pl.ds(o, LANES)]
            sebuf[b][pl.ds(o, LANES)] = plsc.load_gather(srows[b], [o + iota, tv])

        base = ebase + i * CH
        pltpu.sync_copy(sebuf[b], se_h.at[pl.ds(base, CH)])

    @pl.loop(0, per_tile // (2 * CH))
    def _pair(it):
        i0 = it * 2
        load(i0, 0)
        load(i0 + 1, 1)
        extract_store(i0, 0)
        extract_store(i0 + 1, 1)


_sedge_kernel = pl.kernel(
    _sedge_body,
    out_type=jax.ShapeDtypeStruct((EP,), jnp.float32),
    mesh=_MESH,
    scratch_types=[
        pltpu.VMEM((CH,), jnp.int32),
        pltpu.VMEM((CH,), jnp.int32),
        pltpu.VMEM((CH,), jnp.int32),
        pltpu.VMEM((CH,), jnp.int32),
        pltpu.VMEM((CH, 16), jnp.float32),
        pltpu.VMEM((CH, 16), jnp.float32),
        pltpu.VMEM((CH,), jnp.float32),
        pltpu.VMEM((CH,), jnp.float32),
        pltpu.SemaphoreType.DMA,
        pltpu.SemaphoreType.DMA,
    ],
    compiler_params=_SC_PARAMS,
)


def _agg_body(W, dim_split, src_h, dst_h, et_h, x0_h, x1_h, se_h, z_h,
              y0_h, y1_h,
              srcv0, srcv1, dstv0, dstv1, etv0, etv1, gidx0, gidx1,
              sebuf0, sebuf1, rows0, rows1, acc, semx0, semx1):
    srcv = (srcv0, srcv1)
    dstv = (dstv0, dstv1)
    etv = (etv0, etv1)
    gidx = (gidx0, gidx1)
    sebuf = (sebuf0, sebuf1)
    rows = (rows0, rows1)
    semx = (semx0, semx1)
    c = lax.axis_index("c")
    s = lax.axis_index("s")
    rbase = pl.multiple_of(s * TILE_ROWS, 8)
    rows_sl = pl.ds(rbase, TILE_ROWS)
    pltpu.sync_copy(z_h.at[rows_sl], acc.at[rows_sl])
    plsc.subcore_barrier()

    if dim_split:
        per_tile = EP // NS
        ebase = s * per_tile
    else:
        per_tile = EP // (NC * NS)
        ebase = (c * NS + s) * per_tile
    iota = lax.iota(jnp.int32, LANES)

    def load_linear(i, b):
        base = ebase + i * CHA
        pltpu.sync_copy(src_h.at[pl.ds(base, CHA)], srcv[b])
        pltpu.sync_copy(dst_h.at[pl.ds(base, CHA)], dstv[b])
        pltpu.sync_copy(et_h.at[pl.ds(base, CHA)], etv[b])
        pltpu.sync_copy(se_h.at[pl.ds(base, CHA)], sebuf[b])

        @pl.loop(0, CHA // LANES)
        def _mkidx(g):
            o = pl.multiple_of(g * LANES, LANES)
            gidx[b][pl.ds(o, LANES)] = (
                etv[b][pl.ds(o, LANES)] * N + srcv[b][pl.ds(o, LANES)]
            )

    def wait_gather(b):
        pltpu.make_async_copy(x0_h.at[gidx[b]], rows[b], semx[b]).wait()

    def scale_scatter(b):
        @pl.loop(0, CHA // LANES)
        def _scale(g):
            o = pl.multiple_of(g * LANES, LANES)
            sval = sebuf[b][pl.ds(o, LANES)]
            for j in range(LANES):
                sj = _lane_splat(sval, j)
                for q in range(W // LANES):
                    qs = pl.ds(q * LANES, LANES)
                    rows[b][o + j, qs] = rows[b][o + j, qs] * sj

        pltpu.sync_copy(rows[b], acc.at[dstv[b]], add=True)

    def start(b):
        if dim_split:
            @pl.when(c == 0)
            def _():
                pltpu.async_copy(x0_h.at[gidx[b]], rows[b], semx[b])

            @pl.when(c == 1)
            def _():
                pltpu.async_copy(x1_h.at[gidx[b]], rows[b], semx[b])
        else:
            pltpu.async_copy(x0_h.at[gidx[b]], rows[b], semx[b])

    npairs = per_tile // (2 * CHA)
    load_linear(0, 0)
    start(0)

    @pl.loop(0, npairs)
    def _pair(it):
        i0 = it * 2
        load_linear(i0 + 1, 1)
        wait_gather(0)
        start(1)
        scale_scatter(0)

        @pl.when(it + 1 < npairs)
        def _prefetch():
            load_linear(i0 + 2, 0)
            start(0)

        wait_gather(1)
        scale_scatter(1)

    plsc.subcore_barrier()

    @pl.when(c == 0)
    def _():
        pltpu.sync_copy(acc.at[rows_sl], y0_h.at[rows_sl])

    @pl.when(c == 1)
    def _():
        pltpu.sync_copy(acc.at[rows_sl], y1_h.at[rows_sl])


def _make_agg_kernel(W, dim_split):
    return pl.kernel(
        functools.partial(_agg_body, W, dim_split),
        out_type=[
            jax.ShapeDtypeStruct((NP, W), jnp.float32),
            jax.ShapeDtypeStruct((NP, W), jnp.float32),
        ],
        mesh=_MESH,
        scratch_types=(
            [pltpu.VMEM((CHA,), jnp.int32)] * 8
            + [pltpu.VMEM((CHA,), jnp.float32)] * 2
            + [pltpu.VMEM((CHA, W), jnp.float32)] * 2
            + [
                pltpu.VMEM_SHARED((NP, W), jnp.float32),
                pltpu.SemaphoreType.DMA,
                pltpu.SemaphoreType.DMA,
            ]
        ),
        compiler_params=_SC_PARAMS,
    )


_agg_l1 = _make_agg_kernel(32, True)
_agg_l2 = _make_agg_kernel(32, False)


_BN = 2000


def _xt2_body(x_ref, w_ref, x0_ref, x1_ref):
    p = jnp.dot(x_ref[...], w_ref[0], preferred_element_type=jnp.float32)
    half = p.shape[1] // 2
    x0_ref[...] = p[:, :half]
    x1_ref[...] = p[:, half:]


def _xt1_body(x_ref, w_ref, x0_ref):
    x0_ref[...] = jnp.dot(x_ref[...], w_ref[0],
                          preferred_element_type=jnp.float32)


def _xtables(x, w, dout, split):
    nb = N // _BN
    wout = dout // 2 if split else dout
    nout = 2 if split else 1
    out_spec = pl.BlockSpec((_BN, wout), lambda t, i: (t * (N // _BN) + i, 0))
    res = pl.pallas_call(
        _xt2_body if split else _xt1_body,
        grid=(R, nb),
        in_specs=[
            pl.BlockSpec((_BN, x.shape[1]), lambda t, i: (i, 0)),
            pl.BlockSpec((1, w.shape[1], dout), lambda t, i: (t, 0, 0)),
        ],
        out_specs=[out_spec] * nout,
        out_shape=[jax.ShapeDtypeStruct((R * N, wout), jnp.float32)] * nout,
    )(x, w)
    return res


def _recip_body(c0_ref, c1_ref, s_ref):
    s_ref[...] = 1.0 / jnp.clip(c0_ref[...] + c1_ref[...], 1.0, None)


def _recip(cnt0, cnt1):
    r = pl.pallas_call(
        _recip_body,
        out_shape=jax.ShapeDtypeStruct((NP * 16 // 128, 128), jnp.float32),
    )(cnt0.reshape(NP * 16 // 128, 128), cnt1.reshape(NP * 16 // 128, 128))
    return r.reshape(NP, 16)


def _dense_body(act, concat, x_ref, r_ref, b_ref, y0_ref, y1_ref, o_ref):
    t = jnp.dot(x_ref[...], r_ref[...], preferred_element_type=jnp.float32)
    if concat:
        t = t + b_ref[...] + jnp.concatenate([y0_ref[...], y1_ref[...]], axis=1)
    else:
        t = t + b_ref[...] + y0_ref[...] + y1_ref[...]
    o_ref[...] = act(t)


def _dense(x, root, b, y0, y1, act, concat):
    dout = root.shape[1]
    w = dout // 2 if concat else dout
    nb = N // _BN
    return pl.pallas_call(
        functools.partial(_dense_body, act, concat),
        grid=(nb,),
        in_specs=[
            pl.BlockSpec((_BN, x.shape[1]), lambda i: (i, 0)),
            pl.BlockSpec(root.shape, lambda i: (0, 0)),
            pl.BlockSpec((1, dout), lambda i: (0, 0)),
            pl.BlockSpec((_BN, w), lambda i: (i, 0)),
            pl.BlockSpec((_BN, w), lambda i: (i, 0)),
        ],
        out_specs=pl.BlockSpec((_BN, dout), lambda i: (i, 0)),
        out_shape=jax.ShapeDtypeStruct((N, dout), jnp.float32),
    )(x, root, b.reshape(1, dout), y0, y1)


def kernel(emb, w1, root1, b1, w2, root2, b2, edge_index, edge_type):
    pad = EP - E
    src = jnp.concatenate([edge_index[0], jnp.zeros((pad,), jnp.int32)])
    dst = jnp.concatenate([edge_index[1], jnp.full((pad,), DUMP, jnp.int32)])
    et = jnp.concatenate([edge_type, jnp.zeros((pad,), jnp.int32)])

    z16 = jnp.zeros((NP, 16), jnp.float32)
    z32 = jnp.zeros((NP, 32), jnp.float32)

    cnt0, cnt1 = _count_kernel(dst, et, z16)
    s_tab = _recip(cnt0, cnt1)
    s_e = _sedge_kernel(dst, et, s_tab)

    x10, x11 = _xtables(emb, w1, H, True)
    y10, y11 = _agg_l1(src, dst, et, x10, x11, s_e, z32)
    h = _dense(emb, root1, b1, y10[:N], y11[:N],
               lambda v: jnp.maximum(v, 0.0), True)

    (x2,) = _xtables(h, w2, O, False)
    y20, y21 = _agg_l2(src, dst, et, x2, x2, s_e, z32)
    out = _dense(h, root2, b2, y20[:N], y21[:N], jax.nn.sigmoid, False)
    return out
